# Initial kernel scaffold; baseline (speedup 1.0000x reference)
import jax
import jax.numpy as jnp
from jax.experimental import pallas as pl


def kernel(x, edge_index, W1, b1, g1, be1, m1, v1, W2, b2, g2, be2, m2, v2, W3, b3, g3, be3, m3, v3, pW, pb, cW1, cb1, cW2, cb2):
    raise NotImplementedError("write your pallas kernel here")


# speedup vs baseline: 15.3702x; 15.3702x over previous
import functools

import jax
import jax.numpy as jnp
from jax import lax
from jax.experimental import pallas as pl
from jax.experimental.pallas import tpu as pltpu
from jax.experimental.pallas import tpu_sc as plsc

N = 10000
E = 160000
D = 256
DH = 128

NC = 2
NS = 16
NW = NC * NS

NPAD = 10240
R = 1280
GR = NPAD // R

K = 80
EPT = E // NS
NBT = EPT // K
NROWS = E // K

EPW = E // NW
FULL_IT = EPW // 16
REM = EPW - FULL_IT * 16

STRIPE = NPAD // NS

_EPS = 1e-5


def _sc_mesh():
    return plsc.VectorSubcoreMesh(core_axis_name="c", subcore_axis_name="s")


def _deg_kernel(dst_flat, out, dst_v, acc):
    c = lax.axis_index("c")
    s = lax.axis_index("s")
    w = s * NC + c

    z16 = jnp.zeros((16,), jnp.float32)

    def zero_body(i, _):
        acc[pl.ds(i * 16, 16)] = z16
        return 0

    lax.fori_loop(0, NPAD // 16, zero_body, 0)
    dst_v[pl.ds(EPW, 16)] = jnp.zeros((16,), jnp.int32)
    pltpu.sync_copy(dst_flat.at[pl.ds(w * EPW, EPW)], dst_v.at[pl.ds(0, EPW)])

    ones = jnp.ones((16,), jnp.float32)

    def body(i, _):
        idx = dst_v[pl.ds(i * 16, 16)]
        plsc.addupdate_scatter(acc, [idx], ones)
        return 0

    lax.fori_loop(0, FULL_IT, body, 0)
    rem_mask = lax.broadcasted_iota(jnp.int32, (16,), 0) < REM
    idx = dst_v[pl.ds(FULL_IT * 16, 16)]
    plsc.addupdate_scatter(acc, [idx], ones, mask=rem_mask)

    pltpu.sync_copy(acc, out.at[w])


def _run_deg(dst_flat):
    f = pl.kernel(
        _deg_kernel,
        out_type=jax.ShapeDtypeStruct((NW, NPAD), jnp.float32),
        mesh=_sc_mesh(),
        compiler_params=pltpu.CompilerParams(needs_layout_passes=False),
        scratch_types=[
            pltpu.VMEM((EPW + 16,), jnp.int32),
            pltpu.VMEM((NPAD,), jnp.float32),
        ],
    )
    return f(dst_flat)


def _s_kernel(src_flat, dst_flat, dinv2d, out, src_v, dst_v, dinv_v, acc):
    c = lax.axis_index("c")
    s = lax.axis_index("s")
    w = s * NC + c

    z16 = jnp.zeros((16,), jnp.float32)

    def zero_body(i, _):
        acc[pl.ds(i * 16, 16)] = z16
        return 0

    lax.fori_loop(0, NPAD // 16, zero_body, 0)
    src_v[pl.ds(EPW, 16)] = jnp.zeros((16,), jnp.int32)
    dst_v[pl.ds(EPW, 16)] = jnp.zeros((16,), jnp.int32)
    pltpu.sync_copy(src_flat.at[pl.ds(w * EPW, EPW)], src_v.at[pl.ds(0, EPW)])
    pltpu.sync_copy(dst_flat.at[pl.ds(w * EPW, EPW)], dst_v.at[pl.ds(0, EPW)])
    pltpu.sync_copy(dinv2d, dinv_v)

    def step(i, mask):
        d_idx = dst_v[pl.ds(i * 16, 16)]
        val = plsc.load_gather(
            dinv_v,
            [lax.shift_right_logical(d_idx, 7), jnp.bitwise_and(d_idx, 127)],
            mask=mask,
        )
        s_idx = src_v[pl.ds(i * 16, 16)]
        plsc.addupdate_scatter(acc, [s_idx], val, mask=mask)

    full_mask = lax.broadcasted_iota(jnp.int32, (16,), 0) < 16

    def body(i, _):
        step(i, full_mask)
        return 0

    lax.fori_loop(0, FULL_IT, body, 0)
    rem_mask = lax.broadcasted_iota(jnp.int32, (16,), 0) < REM
    step(FULL_IT, rem_mask)

    pltpu.sync_copy(acc, out.at[w])


def _run_s(src_flat, dst_flat, dinv2d):
    f = pl.kernel(
        _s_kernel,
        out_type=jax.ShapeDtypeStruct((NW, NPAD), jnp.float32),
        mesh=_sc_mesh(),
        compiler_params=pltpu.CompilerParams(needs_layout_passes=False),
        scratch_types=[
            pltpu.VMEM((EPW + 16,), jnp.int32),
            pltpu.VMEM((EPW + 16,), jnp.int32),
            pltpu.VMEM((NPAD // 128, 128), jnp.float32),
            pltpu.VMEM((NPAD,), jnp.float32),
        ],
    )
    return f(src_flat, dst_flat, dinv2d)


def _spmm_kernel(hw_flat, src_rs, dst_rs, out, idxs_v, idxd_v, buf, acc, sem):
    c = lax.axis_index("c")
    s = lax.axis_index("s")

    z16 = jnp.zeros((16,), jnp.float32)

    def zbuf_outer(j, _):
        def zbuf_inner(k, _2):
            buf[j, pl.ds(k * 16, 16)] = z16
            return 0

        lax.fori_loop(0, DH // 16, zbuf_inner, 0)
        return 0

    lax.fori_loop(0, K, zbuf_outer, 0)

    for t in range(STRIPE // K):
        pltpu.sync_copy(buf, acc.at[pl.ds(s * STRIPE + t * K, K)])
    plsc.subcore_barrier()

    pltpu.sync_copy(src_rs.at[s], idxs_v)
    pltpu.sync_copy(dst_rs.at[s], idxd_v)

    offv = jnp.zeros((16,), jnp.int32) + (c * NPAD).astype(jnp.int32)

    def off_outer(j, _):
        def off_inner(k, _2):
            v = idxs_v[j, pl.ds(k * 16, 16)]
            idxs_v[j, pl.ds(k * 16, 16)] = v + offv
            return 0

        lax.fori_loop(0, K // 16, off_inner, 0)
        return 0

    lax.fori_loop(0, NBT, off_outer, 0)

    def main(j, _):
        pltpu.async_copy(hw_flat.at[idxs_v.at[j]], buf, sem).wait()
        pltpu.sync_copy(buf, acc.at[idxd_v.at[j]], add=True)
        return 0

    lax.fori_loop(0, NBT, main, 0)
    plsc.subcore_barrier()

    pltpu.sync_copy(
        acc.at[pl.ds(s * STRIPE, STRIPE)],
        out.at[pl.ds(c * NPAD + s * STRIPE, STRIPE)],
    )


def _run_spmm(hw_flat, src_rs, dst_rs):
    f = pl.kernel(
        _spmm_kernel,
        out_type=jax.ShapeDtypeStruct((2 * NPAD, DH), jnp.float32),
        mesh=_sc_mesh(),
        compiler_params=pltpu.CompilerParams(needs_layout_passes=False),
        scratch_types=[
            pltpu.VMEM((NBT, K), jnp.int32),
            pltpu.VMEM((NBT, K), jnp.int32),
            pltpu.VMEM((K, DH), jnp.float32),
            pltpu.VMEM_SHARED((NPAD, DH), jnp.float32),
            pltpu.SemaphoreType.DMA,
        ],
    )
    return f(hw_flat, src_rs, dst_rs)


def _hw1_body(deg_ref, x_ref, w_ref, hwp_ref, dinvc_ref):
    deg = jnp.sum(deg_ref[...], axis=0) + 1.0
    dinv_col = lax.rsqrt(deg).reshape(R, 1)
    hw = jnp.dot(x_ref[...], w_ref[...], preferred_element_type=jnp.float32)
    hwp_ref[0] = hw * dinv_col
    dinvc_ref[...] = jnp.broadcast_to(dinv_col, (R, DH))


def _run_hw1(deg_part, x, W1):
    return pl.pallas_call(
        _hw1_body,
        grid=(2, GR),
        in_specs=[
            pl.BlockSpec((NW, R), lambda h, r: (0, r)),
            pl.BlockSpec((R, D), lambda h, r: (r, 0)),
            pl.BlockSpec((D, DH), lambda h, r: (0, h)),
        ],
        out_specs=[
            pl.BlockSpec((1, R, DH), lambda h, r: (h, r, 0)),
            pl.BlockSpec((R, DH), lambda h, r: (r, 0)),
        ],
        out_shape=[
            jax.ShapeDtypeStruct((2, NPAD, DH), jnp.float32),
            jax.ShapeDtypeStruct((NPAD, DH), jnp.float32),
        ],
    )(deg_part, x, W1)


def _dinv2d_body(deg_ref, out_ref):
    deg = jnp.sum(deg_ref[...], axis=0, keepdims=True) + 1.0
    out_ref[...] = lax.rsqrt(deg).reshape(NPAD // 128, 128)


def _run_dinv2d(deg_part):
    return pl.pallas_call(
        _dinv2d_body,
        out_shape=jax.ShapeDtypeStruct((NPAD // 128, 128), jnp.float32),
    )(deg_part)


def _mid_body(scat_ref, hwp_ref, dinvc_ref, al_ref, be_ref, w2_ref, out_ref):
    dv = dinvc_ref[...]
    acc = None
    for hh in range(2):
        t = (scat_ref[hh] + hwp_ref[hh]) * dv
        h1 = jnp.maximum(t * al_ref[hh] + be_ref[hh], 0.0)
        p = jnp.dot(h1, w2_ref[hh], preferred_element_type=jnp.float32)
        acc = p if acc is None else acc + p
    out_ref[0] = acc * dv


def _run_mid(scat1, hwp1, dinvc, al1, be1, W2_rs):
    return pl.pallas_call(
        _mid_body,
        grid=(2, GR),
        in_specs=[
            pl.BlockSpec((2, R, DH), lambda h, r: (0, r, 0)),
            pl.BlockSpec((2, R, DH), lambda h, r: (0, r, 0)),
            pl.BlockSpec((R, DH), lambda h, r: (r, 0)),
            pl.BlockSpec((2, 128), lambda h, r: (0, 0)),
            pl.BlockSpec((2, 128), lambda h, r: (0, 0)),
            pl.BlockSpec((2, 128, 128), lambda h, r: (0, 0, h)),
        ],
        out_specs=pl.BlockSpec((1, R, DH), lambda h, r: (h, r, 0)),
        out_shape=jax.ShapeDtypeStruct((2, NPAD, DH), jnp.float32),
    )(scat1, hwp1, dinvc, al1, be1, W2_rs)


def _final_body(scat_ref, hwp_ref, dinvc_ref, al_ref, be_ref, deg_ref, s_ref,
                al3_ref, be3_ref, w3_ref, pw_ref, pb_ref, cw1_ref, cb1_ref,
                cw2_ref, cb2_ref, out_ref, acc_ref):
    r = pl.program_id(0)
    dv = dinvc_ref[...]
    row_ids = lax.broadcasted_iota(jnp.int32, (R, 1), 0) + r * R
    row_ok = row_ids < N

    parts = []
    for hh in range(2):
        t = (scat_ref[hh] + hwp_ref[hh]) * dv
        h2 = jnp.maximum(t * al_ref[hh] + be_ref[hh], 0.0)
        h2 = jnp.where(row_ok, h2, 0.0)
        parts.append(h2)

    deg_row = jnp.sum(deg_ref[...], axis=0, keepdims=True) + 1.0
    dinv_row = lax.rsqrt(deg_row)
    s_row = jnp.sum(s_ref[...], axis=0, keepdims=True)
    col_ids = lax.broadcasted_iota(jnp.int32, (1, R), 1) + r * R
    u = jnp.where(col_ids < N, dinv_row * (s_row + dinv_row), 0.0)
    u = u * (1.0 / N)

    part = jnp.concatenate(
        [jnp.dot(u, p, preferred_element_type=jnp.float32) for p in parts],
        axis=1,
    )

    @pl.when(r == 0)
    def _():
        acc_ref[...] = part

    @pl.when(r > 0)
    def _():
        acc_ref[...] = acc_ref[...] + part

    @pl.when(r == GR - 1)
    def _():
        pooled = acc_ref[...]
        y = jnp.dot(pooled, w3_ref[...], preferred_element_type=jnp.float32)
        z = y * al3_ref[...] + be3_ref[...]
        p = jnp.maximum(
            jnp.dot(z, pw_ref[...], preferred_element_type=jnp.float32)
            + pb_ref[...], 0.0)
        cc = jnp.maximum(
            jnp.dot(p, cw1_ref[...], preferred_element_type=jnp.float32)
            + cb1_ref[...], 0.0)
        out_ref[...] = (
            jnp.dot(cc, cw2_ref[...], preferred_element_type=jnp.float32)
            + cb2_ref[...])


def _run_final(scat2, hwp2, dinvc, al2, be2, deg_part, s_part, al3, be3,
               W3, pW, pb, cW1, cb1, cW2, cb2):
    def full(shape):
        nz = len(shape)
        return pl.BlockSpec(shape, lambda r, _n=nz: (0,) * _n)

    return pl.pallas_call(
        _final_body,
        grid=(GR,),
        in_specs=[
            pl.BlockSpec((2, R, DH), lambda r: (0, r, 0)),
            pl.BlockSpec((2, R, DH), lambda r: (0, r, 0)),
            pl.BlockSpec((R, DH), lambda r: (r, 0)),
            full((2, 128)),
            full((2, 128)),
            pl.BlockSpec((NW, R), lambda r: (0, r)),
            pl.BlockSpec((NW, R), lambda r: (0, r)),
            full((1, D)),
            full((1, D)),
            full((D, D)),
            full((D, DH)),
            full((1, DH)),
            full((DH, 64)),
            full((1, 64)),
            full((64, DH)),
            full((1, DH)),
        ],
        out_specs=pl.BlockSpec((1, DH), lambda r: (0, 0)),
        out_shape=jax.ShapeDtypeStruct((1, DH), jnp.float32),
        scratch_shapes=[pltpu.VMEM((1, D), jnp.float32)],
    )(scat2, hwp2, dinvc, al2, be2, deg_part, s_part, al3, be3,
      W3, pW, pb, cW1, cb1, cW2, cb2)


def kernel(x, edge_index, W1, b1, g1, be1, m1, v1, W2, b2, g2, be2, m2, v2,
           W3, b3, g3, be3, m3, v3, pW, pb, cW1, cb1, cW2, cb2):
    def fold(b, g, be, m, v):
        scale = g * lax.rsqrt(v + _EPS)
        shift = b * scale + (be - m * scale)
        return scale, shift

    al1, bp1 = fold(b1, g1, be1, m1, v1)
    al2, bp2 = fold(b2, g2, be2, m2, v2)
    al3, bp3 = fold(b3, g3, be3, m3, v3)

    al1 = al1.reshape(2, 128)
    bp1 = bp1.reshape(2, 128)
    al2 = al2.reshape(2, 128)
    bp2 = bp2.reshape(2, 128)
    al3 = al3.reshape(1, D)
    bp3 = bp3.reshape(1, D)

    W2_rs = W2.reshape(2, 128, D)

    src_flat = edge_index[0]
    dst_flat = edge_index[1]
    src_rs = src_flat.reshape(NS, NBT, K)
    dst_rs = dst_flat.reshape(NS, NBT, K)

    deg_part = _run_deg(dst_flat)
    hwp1, dinvc = _run_hw1(deg_part, x, W1)
    dinv2d = _run_dinv2d(deg_part)
    s_part = _run_s(src_flat, dst_flat, dinv2d)
    scat1 = _run_spmm(hwp1.reshape(2 * NPAD, DH), src_rs, dst_rs)
    hwp2 = _run_mid(scat1.reshape(2, NPAD, DH), hwp1, dinvc, al1, bp1, W2_rs)
    scat2 = _run_spmm(hwp2.reshape(2 * NPAD, DH), src_rs, dst_rs)
    out = _run_final(scat2.reshape(2, NPAD, DH), hwp2, dinvc, al2, bp2,
                     deg_part, s_part, al3, bp3, W3, pW,
                     pb.reshape(1, DH), cW1, cb1.reshape(1, 64),
                     cW2, cb2.reshape(1, DH))
    return out


# speedup vs baseline: 22.4749x; 1.4622x over previous
import functools

import jax
import jax.numpy as jnp
from jax import lax
from jax.experimental import pallas as pl
from jax.experimental.pallas import tpu as pltpu
from jax.experimental.pallas import tpu_sc as plsc

N = 10000
E = 160000
D = 256
DH = 128

NC = 2
NS = 16
NW = NC * NS

NPAD = 10240
R = 1280
GR = NPAD // R

K = 80
EPT = E // NS
NBT = EPT // K
NCH = 5
CH_R = NBT // NCH

EPW = E // NW
FULL_IT = EPW // 16
REM = EPW - FULL_IT * 16

STRIPE = NPAD // NS

_EPS = 1e-5


def _sc_mesh():
    return plsc.VectorSubcoreMesh(core_axis_name="c", subcore_axis_name="s")


def _deg_kernel(dst_flat, out, dst_v, acc):
    c = lax.axis_index("c")
    s = lax.axis_index("s")
    w = s * NC + c

    z16 = jnp.zeros((16,), jnp.float32)

    def zero_body(i, _):
        acc[pl.ds(i * 16, 16)] = z16
        return 0

    lax.fori_loop(0, NPAD // 16, zero_body, 0)
    dst_v[pl.ds(EPW, 16)] = jnp.zeros((16,), jnp.int32)
    pltpu.sync_copy(dst_flat.at[pl.ds(w * EPW, EPW)], dst_v.at[pl.ds(0, EPW)])

    ones = jnp.ones((16,), jnp.float32)

    def body(i, _):
        idx = dst_v[pl.ds(i * 16, 16)]
        plsc.addupdate_scatter(acc, [idx], ones)
        return 0

    lax.fori_loop(0, FULL_IT, body, 0)
    rem_mask = lax.broadcasted_iota(jnp.int32, (16,), 0) < REM
    idx = dst_v[pl.ds(FULL_IT * 16, 16)]
    plsc.addupdate_scatter(acc, [idx], ones, mask=rem_mask)

    pltpu.sync_copy(acc, out.at[w])


def _run_deg(dst_flat):
    f = pl.kernel(
        _deg_kernel,
        out_type=jax.ShapeDtypeStruct((NW, NPAD), jnp.float32),
        mesh=_sc_mesh(),
        compiler_params=pltpu.CompilerParams(needs_layout_passes=False),
        scratch_types=[
            pltpu.VMEM((EPW + 16,), jnp.int32),
            pltpu.VMEM((NPAD,), jnp.float32),
        ],
    )
    return f(dst_flat)


def _s_kernel(src_flat, dst_flat, dinv2d, out, src_v, dst_v, dinv_v, acc):
    c = lax.axis_index("c")
    s = lax.axis_index("s")
    w = s * NC + c

    z16 = jnp.zeros((16,), jnp.float32)

    def zero_body(i, _):
        acc[pl.ds(i * 16, 16)] = z16
        return 0

    lax.fori_loop(0, NPAD // 16, zero_body, 0)
    src_v[pl.ds(EPW, 16)] = jnp.zeros((16,), jnp.int32)
    dst_v[pl.ds(EPW, 16)] = jnp.zeros((16,), jnp.int32)
    pltpu.sync_copy(src_flat.at[pl.ds(w * EPW, EPW)], src_v.at[pl.ds(0, EPW)])
    pltpu.sync_copy(dst_flat.at[pl.ds(w * EPW, EPW)], dst_v.at[pl.ds(0, EPW)])
    pltpu.sync_copy(dinv2d, dinv_v)

    def step(i, mask):
        d_idx = dst_v[pl.ds(i * 16, 16)]
        val = plsc.load_gather(
            dinv_v,
            [lax.shift_right_logical(d_idx, 7), jnp.bitwise_and(d_idx, 127)],
            mask=mask,
        )
        s_idx = src_v[pl.ds(i * 16, 16)]
        plsc.addupdate_scatter(acc, [s_idx], val, mask=mask)

    full_mask = lax.broadcasted_iota(jnp.int32, (16,), 0) < 16

    def body(i, _):
        step(i, full_mask)
        return 0

    lax.fori_loop(0, FULL_IT, body, 0)
    rem_mask = lax.broadcasted_iota(jnp.int32, (16,), 0) < REM
    step(FULL_IT, rem_mask)

    pltpu.sync_copy(acc, out.at[w])


def _run_s(src_flat, dst_flat, dinv2d):
    f = pl.kernel(
        _s_kernel,
        out_type=jax.ShapeDtypeStruct((NW, NPAD), jnp.float32),
        mesh=_sc_mesh(),
        compiler_params=pltpu.CompilerParams(needs_layout_passes=False),
        scratch_types=[
            pltpu.VMEM((EPW + 16,), jnp.int32),
            pltpu.VMEM((EPW + 16,), jnp.int32),
            pltpu.VMEM((NPAD // 128, 128), jnp.float32),
            pltpu.VMEM((NPAD,), jnp.float32),
        ],
    )
    return f(src_flat, dst_flat, dinv2d)


def _spmm_kernel(hw_flat, src_rs, dst_rs, out, idxs_v, idxd_v, buf0, buf1,
                 acc, sems2):
    c = lax.axis_index("c")
    s = lax.axis_index("s")

    z16 = jnp.zeros((16,), jnp.float32)

    def zbuf_outer(j, _):
        def zbuf_inner(k, _2):
            buf0[j, pl.ds(k * 16, 16)] = z16
            return 0

        lax.fori_loop(0, DH // 16, zbuf_inner, 0)
        return 0

    lax.fori_loop(0, K, zbuf_outer, 0)

    for t in range(STRIPE // K):
        pltpu.sync_copy(buf0, acc.at[pl.ds(s * STRIPE + t * K, K)])
    plsc.subcore_barrier()

    offv = jnp.zeros((16,), jnp.int32) + (c * NPAD).astype(jnp.int32)
    bufs = (buf0, buf1)
    sems = (sems2.at[0], sems2.at[1])

    for ch in range(NCH):
        pltpu.sync_copy(src_rs.at[s, ch], idxs_v)
        pltpu.sync_copy(dst_rs.at[s, ch], idxd_v)

        def off_outer(j, _):
            def off_inner(k, _2):
                v = idxs_v[j, pl.ds(k * 16, 16)]
                idxs_v[j, pl.ds(k * 16, 16)] = v + offv
                return 0

            lax.fori_loop(0, K // 16, off_inner, 0)
            return 0

        lax.fori_loop(0, CH_R, off_outer, 0)

        pltpu.async_copy(hw_flat.at[idxs_v.at[0]], buf0, sems[0])

        @pl.loop(0, CH_R - 1, step=2)
        def _(j):
            for b in range(2):
                nxt = j + b + 1

                @pl.when(nxt < CH_R)
                def _():
                    pltpu.async_copy(
                        hw_flat.at[idxs_v.at[nxt]], bufs[1 - b], sems[1 - b])

                pltpu.make_async_copy(
                    hw_flat.at[idxs_v.at[j + b]], bufs[b], sems[b]).wait()
                pltpu.sync_copy(bufs[b], acc.at[idxd_v.at[j + b]], add=True)

        pltpu.make_async_copy(
            hw_flat.at[idxs_v.at[CH_R - 1]], buf0, sems[0]).wait()
        pltpu.sync_copy(buf0, acc.at[idxd_v.at[CH_R - 1]], add=True)

    plsc.subcore_barrier()

    pltpu.sync_copy(
        acc.at[pl.ds(s * STRIPE, STRIPE)],
        out.at[pl.ds(c * NPAD + s * STRIPE, STRIPE)],
    )


def _run_spmm(hw_flat, src_rs, dst_rs):
    f = pl.kernel(
        _spmm_kernel,
        out_type=jax.ShapeDtypeStruct((2 * NPAD, DH), jnp.float32),
        mesh=_sc_mesh(),
        compiler_params=pltpu.CompilerParams(needs_layout_passes=False),
        scratch_types=[
            pltpu.VMEM((CH_R, K), jnp.int32),
            pltpu.VMEM((CH_R, K), jnp.int32),
            pltpu.VMEM((K, DH), jnp.float32),
            pltpu.VMEM((K, DH), jnp.float32),
            pltpu.VMEM_SHARED((NPAD, DH), jnp.float32),
            pltpu.SemaphoreType.DMA((2,)),
        ],
    )
    return f(hw_flat, src_rs, dst_rs)


def _hw1_body(deg_ref, x_ref, w_ref, hwp_ref, dinvc_ref):
    deg = jnp.sum(deg_ref[...], axis=0) + 1.0
    dinv_col = lax.rsqrt(deg).reshape(R, 1)
    hw = jnp.dot(x_ref[...], w_ref[...], preferred_element_type=jnp.float32)
    hwp_ref[0] = hw * dinv_col
    dinvc_ref[...] = jnp.broadcast_to(dinv_col, (R, DH))


def _run_hw1(deg_part, x, W1):
    return pl.pallas_call(
        _hw1_body,
        grid=(2, GR),
        in_specs=[
            pl.BlockSpec((NW, R), lambda h, r: (0, r)),
            pl.BlockSpec((R, D), lambda h, r: (r, 0)),
            pl.BlockSpec((D, DH), lambda h, r: (0, h)),
        ],
        out_specs=[
            pl.BlockSpec((1, R, DH), lambda h, r: (h, r, 0)),
            pl.BlockSpec((R, DH), lambda h, r: (r, 0)),
        ],
        out_shape=[
            jax.ShapeDtypeStruct((2, NPAD, DH), jnp.float32),
            jax.ShapeDtypeStruct((NPAD, DH), jnp.float32),
        ],
    )(deg_part, x, W1)


def _dinv2d_body(deg_ref, out_ref):
    deg = jnp.sum(deg_ref[...], axis=0, keepdims=True) + 1.0
    out_ref[...] = lax.rsqrt(deg).reshape(NPAD // 128, 128)


def _run_dinv2d(deg_part):
    return pl.pallas_call(
        _dinv2d_body,
        out_shape=jax.ShapeDtypeStruct((NPAD // 128, 128), jnp.float32),
    )(deg_part)


def _mid_body(scat_ref, hwp_ref, dinvc_ref, al_ref, be_ref, w2_ref, out_ref):
    dv = dinvc_ref[...]
    acc = None
    for hh in range(2):
        t = (scat_ref[hh] + hwp_ref[hh]) * dv
        h1 = jnp.maximum(t * al_ref[hh] + be_ref[hh], 0.0)
        p = jnp.dot(h1, w2_ref[hh], preferred_element_type=jnp.float32)
        acc = p if acc is None else acc + p
    out_ref[0] = acc * dv


def _run_mid(scat1, hwp1, dinvc, al1, be1, W2_rs):
    return pl.pallas_call(
        _mid_body,
        grid=(2, GR),
        in_specs=[
            pl.BlockSpec((2, R, DH), lambda h, r: (0, r, 0)),
            pl.BlockSpec((2, R, DH), lambda h, r: (0, r, 0)),
            pl.BlockSpec((R, DH), lambda h, r: (r, 0)),
            pl.BlockSpec((2, 128), lambda h, r: (0, 0)),
            pl.BlockSpec((2, 128), lambda h, r: (0, 0)),
            pl.BlockSpec((2, 128, 128), lambda h, r: (0, 0, h)),
        ],
        out_specs=pl.BlockSpec((1, R, DH), lambda h, r: (h, r, 0)),
        out_shape=jax.ShapeDtypeStruct((2, NPAD, DH), jnp.float32),
    )(scat1, hwp1, dinvc, al1, be1, W2_rs)


def _final_body(scat_ref, hwp_ref, dinvc_ref, al_ref, be_ref, deg_ref, s_ref,
                al3_ref, be3_ref, w3_ref, pw_ref, pb_ref, cw1_ref, cb1_ref,
                cw2_ref, cb2_ref, out_ref, acc_ref):
    r = pl.program_id(0)
    dv = dinvc_ref[...]
    row_ids = lax.broadcasted_iota(jnp.int32, (R, 1), 0) + r * R
    row_ok = row_ids < N

    parts = []
    for hh in range(2):
        t = (scat_ref[hh] + hwp_ref[hh]) * dv
        h2 = jnp.maximum(t * al_ref[hh] + be_ref[hh], 0.0)
        h2 = jnp.where(row_ok, h2, 0.0)
        parts.append(h2)

    deg_row = jnp.sum(deg_ref[...], axis=0, keepdims=True) + 1.0
    dinv_row = lax.rsqrt(deg_row)
    s_row = jnp.sum(s_ref[...], axis=0, keepdims=True)
    col_ids = lax.broadcasted_iota(jnp.int32, (1, R), 1) + r * R
    u = jnp.where(col_ids < N, dinv_row * (s_row + dinv_row), 0.0)
    u = u * (1.0 / N)

    part = jnp.concatenate(
        [jnp.dot(u, p, preferred_element_type=jnp.float32) for p in parts],
        axis=1,
    )

    @pl.when(r == 0)
    def _():
        acc_ref[...] = part

    @pl.when(r > 0)
    def _():
        acc_ref[...] = acc_ref[...] + part

    @pl.when(r == GR - 1)
    def _():
        pooled = acc_ref[...]
        y = jnp.dot(pooled, w3_ref[...], preferred_element_type=jnp.float32)
        z = y * al3_ref[...] + be3_ref[...]
        p = jnp.maximum(
            jnp.dot(z, pw_ref[...], preferred_element_type=jnp.float32)
            + pb_ref[...], 0.0)
        cc = jnp.maximum(
            jnp.dot(p, cw1_ref[...], preferred_element_type=jnp.float32)
            + cb1_ref[...], 0.0)
        out_ref[...] = (
            jnp.dot(cc, cw2_ref[...], preferred_element_type=jnp.float32)
            + cb2_ref[...])


def _run_final(scat2, hwp2, dinvc, al2, be2, deg_part, s_part, al3, be3,
               W3, pW, pb, cW1, cb1, cW2, cb2):
    def full(shape):
        nz = len(shape)
        return pl.BlockSpec(shape, lambda r, _n=nz: (0,) * _n)

    return pl.pallas_call(
        _final_body,
        grid=(GR,),
        in_specs=[
            pl.BlockSpec((2, R, DH), lambda r: (0, r, 0)),
            pl.BlockSpec((2, R, DH), lambda r: (0, r, 0)),
            pl.BlockSpec((R, DH), lambda r: (r, 0)),
            full((2, 128)),
            full((2, 128)),
            pl.BlockSpec((NW, R), lambda r: (0, r)),
            pl.BlockSpec((NW, R), lambda r: (0, r)),
            full((1, D)),
            full((1, D)),
            full((D, D)),
            full((D, DH)),
            full((1, DH)),
            full((DH, 64)),
            full((1, 64)),
            full((64, DH)),
            full((1, DH)),
        ],
        out_specs=pl.BlockSpec((1, DH), lambda r: (0, 0)),
        out_shape=jax.ShapeDtypeStruct((1, DH), jnp.float32),
        scratch_shapes=[pltpu.VMEM((1, D), jnp.float32)],
    )(scat2, hwp2, dinvc, al2, be2, deg_part, s_part, al3, be3,
      W3, pW, pb, cW1, cb1, cW2, cb2)


def kernel(x, edge_index, W1, b1, g1, be1, m1, v1, W2, b2, g2, be2, m2, v2,
           W3, b3, g3, be3, m3, v3, pW, pb, cW1, cb1, cW2, cb2):
    def fold(b, g, be, m, v):
        scale = g * lax.rsqrt(v + _EPS)
        shift = b * scale + (be - m * scale)
        return scale, shift

    al1, bp1 = fold(b1, g1, be1, m1, v1)
    al2, bp2 = fold(b2, g2, be2, m2, v2)
    al3, bp3 = fold(b3, g3, be3, m3, v3)

    al1 = al1.reshape(2, 128)
    bp1 = bp1.reshape(2, 128)
    al2 = al2.reshape(2, 128)
    bp2 = bp2.reshape(2, 128)
    al3 = al3.reshape(1, D)
    bp3 = bp3.reshape(1, D)

    W2_rs = W2.reshape(2, 128, D)

    src_flat = edge_index[0]
    dst_flat = edge_index[1]
    src_rs = src_flat.reshape(NS, NCH, CH_R, K)
    dst_rs = dst_flat.reshape(NS, NCH, CH_R, K)

    deg_part = _run_deg(dst_flat)
    hwp1, dinvc = _run_hw1(deg_part, x, W1)
    dinv2d = _run_dinv2d(deg_part)
    s_part = _run_s(src_flat, dst_flat, dinv2d)
    scat1 = _run_spmm(hwp1.reshape(2 * NPAD, DH), src_rs, dst_rs)
    hwp2 = _run_mid(scat1.reshape(2, NPAD, DH), hwp1, dinvc, al1, bp1, W2_rs)
    scat2 = _run_spmm(hwp2.reshape(2 * NPAD, DH), src_rs, dst_rs)
    out = _run_final(scat2.reshape(2, NPAD, DH), hwp2, dinvc, al2, bp2,
                     deg_part, s_part, al3, bp3, W3, pW,
                     pb.reshape(1, DH), cW1, cb1.reshape(1, 64),
                     cW2, cb2.reshape(1, DH))
    return out


# speedup vs baseline: 22.4946x; 1.0009x over previous
import functools

import jax
import jax.numpy as jnp
from jax import lax
from jax.experimental import pallas as pl
from jax.experimental.pallas import tpu as pltpu
from jax.experimental.pallas import tpu_sc as plsc

N = 10000
E = 160000
D = 256
DH = 128

NC = 2
NS = 16
NW = NC * NS

NPAD = 10240
R = 1280
GR = NPAD // R

K = 128
EPTP = 10240
EP = NS * EPTP
NBT = EPTP // K
NCH = 5
CH_R = NBT // NCH

EPW = E // NW
FULL_IT = EPW // 16
REM = EPW - FULL_IT * 16

STRIPE = NPAD // NS

_EPS = 1e-5


def _sc_mesh():
    return plsc.VectorSubcoreMesh(core_axis_name="c", subcore_axis_name="s")


def _deg_kernel(dst_flat, out, dst_v, acc):
    c = lax.axis_index("c")
    s = lax.axis_index("s")
    w = s * NC + c

    z16 = jnp.zeros((16,), jnp.float32)

    def zero_body(i, _):
        acc[pl.ds(i * 16, 16)] = z16
        return 0

    lax.fori_loop(0, NPAD // 16, zero_body, 0)
    dst_v[pl.ds(EPW, 16)] = jnp.zeros((16,), jnp.int32)
    pltpu.sync_copy(dst_flat.at[pl.ds(w * EPW, EPW)], dst_v.at[pl.ds(0, EPW)])

    ones = jnp.ones((16,), jnp.float32)

    def body(i, _):
        idx = dst_v[pl.ds(i * 16, 16)]
        plsc.addupdate_scatter(acc, [idx], ones)
        return 0

    lax.fori_loop(0, FULL_IT, body, 0)
    rem_mask = lax.broadcasted_iota(jnp.int32, (16,), 0) < REM
    idx = dst_v[pl.ds(FULL_IT * 16, 16)]
    plsc.addupdate_scatter(acc, [idx], ones, mask=rem_mask)

    pltpu.sync_copy(acc, out.at[w])


def _run_deg(dst_flat):
    f = pl.kernel(
        _deg_kernel,
        out_type=jax.ShapeDtypeStruct((NW, NPAD), jnp.float32),
        mesh=_sc_mesh(),
        compiler_params=pltpu.CompilerParams(needs_layout_passes=False),
        scratch_types=[
            pltpu.VMEM((EPW + 16,), jnp.int32),
            pltpu.VMEM((NPAD,), jnp.float32),
        ],
    )
    return f(dst_flat)


def _s_kernel(src_flat, dst_flat, dinv2d, out, src_v, dst_v, dinv_v, acc):
    c = lax.axis_index("c")
    s = lax.axis_index("s")
    w = s * NC + c

    z16 = jnp.zeros((16,), jnp.float32)

    def zero_body(i, _):
        acc[pl.ds(i * 16, 16)] = z16
        return 0

    lax.fori_loop(0, NPAD // 16, zero_body, 0)
    src_v[pl.ds(EPW, 16)] = jnp.zeros((16,), jnp.int32)
    dst_v[pl.ds(EPW, 16)] = jnp.zeros((16,), jnp.int32)
    pltpu.sync_copy(src_flat.at[pl.ds(w * EPW, EPW)], src_v.at[pl.ds(0, EPW)])
    pltpu.sync_copy(dst_flat.at[pl.ds(w * EPW, EPW)], dst_v.at[pl.ds(0, EPW)])
    pltpu.sync_copy(dinv2d, dinv_v)

    def step(i, mask):
        d_idx = dst_v[pl.ds(i * 16, 16)]
        val = plsc.load_gather(
            dinv_v,
            [lax.shift_right_logical(d_idx, 7), jnp.bitwise_and(d_idx, 127)],
            mask=mask,
        )
        s_idx = src_v[pl.ds(i * 16, 16)]
        plsc.addupdate_scatter(acc, [s_idx], val, mask=mask)

    full_mask = lax.broadcasted_iota(jnp.int32, (16,), 0) < 16

    def body(i, _):
        step(i, full_mask)
        return 0

    lax.fori_loop(0, FULL_IT, body, 0)
    rem_mask = lax.broadcasted_iota(jnp.int32, (16,), 0) < REM
    step(FULL_IT, rem_mask)

    pltpu.sync_copy(acc, out.at[w])


def _run_s(src_flat, dst_flat, dinv2d):
    f = pl.kernel(
        _s_kernel,
        out_type=jax.ShapeDtypeStruct((NW, NPAD), jnp.float32),
        mesh=_sc_mesh(),
        compiler_params=pltpu.CompilerParams(needs_layout_passes=False),
        scratch_types=[
            pltpu.VMEM((EPW + 16,), jnp.int32),
            pltpu.VMEM((EPW + 16,), jnp.int32),
            pltpu.VMEM((NPAD // 128, 128), jnp.float32),
            pltpu.VMEM((NPAD,), jnp.float32),
        ],
    )
    return f(src_flat, dst_flat, dinv2d)


def _spmm_kernel(hw_flat, src_rs, dst_rs, out, idxs_v, idxd_v, buf0, buf1,
                 acc, sems2):
    c = lax.axis_index("c")
    s = lax.axis_index("s")

    z16 = jnp.zeros((16,), jnp.float32)

    def zbuf_outer(j, _):
        def zbuf_inner(k, _2):
            buf0[j, pl.ds(k * 16, 16)] = z16
            return 0

        lax.fori_loop(0, DH // 16, zbuf_inner, 0)
        return 0

    lax.fori_loop(0, K, zbuf_outer, 0)

    for t in range(STRIPE // K):
        pltpu.sync_copy(buf0, acc.at[pl.ds(s * STRIPE + t * K, K)])
    plsc.subcore_barrier()

    offv = jnp.zeros((16,), jnp.int32) + (c * NPAD).astype(jnp.int32)
    bufs = (buf0, buf1)
    sems = (sems2.at[0], sems2.at[1])

    for ch in range(NCH):
        pltpu.sync_copy(src_rs.at[s, ch], idxs_v)
        pltpu.sync_copy(dst_rs.at[s, ch], idxd_v)

        def off_outer(j, _):
            def off_inner(k, _2):
                v = idxs_v[j, pl.ds(k * 16, 16)]
                idxs_v[j, pl.ds(k * 16, 16)] = v + offv
                return 0

            lax.fori_loop(0, K // 16, off_inner, 0)
            return 0

        lax.fori_loop(0, CH_R, off_outer, 0)

        pltpu.async_copy(hw_flat.at[idxs_v.at[0]], buf0, sems[0])

        @pl.loop(0, CH_R, step=2)
        def _(j):
            for b in range(2):
                nxt = j + b + 1

                @pl.when(nxt < CH_R)
                def _():
                    pltpu.async_copy(
                        hw_flat.at[idxs_v.at[nxt]], bufs[1 - b], sems[1 - b])

                pltpu.make_async_copy(
                    hw_flat.at[idxs_v.at[j + b]], bufs[b], sems[b]).wait()
                pltpu.sync_copy(bufs[b], acc.at[idxd_v.at[j + b]], add=True)

    plsc.subcore_barrier()

    pltpu.sync_copy(
        acc.at[pl.ds(s * STRIPE, STRIPE)],
        out.at[pl.ds(c * NPAD + s * STRIPE, STRIPE)],
    )


def _run_spmm(hw_flat, src_rs, dst_rs):
    f = pl.kernel(
        _spmm_kernel,
        out_type=jax.ShapeDtypeStruct((2 * NPAD, DH), jnp.float32),
        mesh=_sc_mesh(),
        compiler_params=pltpu.CompilerParams(needs_layout_passes=False),
        scratch_types=[
            pltpu.VMEM((CH_R, K), jnp.int32),
            pltpu.VMEM((CH_R, K), jnp.int32),
            pltpu.VMEM((K, DH), jnp.float32),
            pltpu.VMEM((K, DH), jnp.float32),
            pltpu.VMEM_SHARED((NPAD, DH), jnp.float32),
            pltpu.SemaphoreType.DMA((2,)),
        ],
    )
    return f(hw_flat, src_rs, dst_rs)


def _hw1_body(deg_ref, x_ref, w_ref, hwp_ref, dinvc_ref, dinv2d_ref):
    deg = jnp.sum(deg_ref[...], axis=0) + 1.0
    dinv_col = lax.rsqrt(deg).reshape(R, 1)
    hw = jnp.dot(x_ref[...], w_ref[...], preferred_element_type=jnp.float32, precision=lax.Precision.HIGHEST)
    hwp_ref[0] = hw * dinv_col
    dinvc_ref[...] = jnp.broadcast_to(dinv_col, (R, DH))
    dinv2d_ref[...] = dinv_col.reshape(1, R // 128, 128)


def _run_hw1(deg_part, x, W1):
    return pl.pallas_call(
        _hw1_body,
        grid=(2, GR),
        in_specs=[
            pl.BlockSpec((NW, R), lambda h, r: (0, r)),
            pl.BlockSpec((R, D), lambda h, r: (r, 0)),
            pl.BlockSpec((D, DH), lambda h, r: (0, h)),
        ],
        out_specs=[
            pl.BlockSpec((1, R, DH), lambda h, r: (h, r, 0)),
            pl.BlockSpec((R, DH), lambda h, r: (r, 0)),
            pl.BlockSpec((1, R // 128, 128), lambda h, r: (r, 0, 0)),
        ],
        out_shape=[
            jax.ShapeDtypeStruct((2, NPAD, DH), jnp.float32),
            jax.ShapeDtypeStruct((NPAD, DH), jnp.float32),
            jax.ShapeDtypeStruct((GR, R // 128, 128), jnp.float32),
        ],
    )(deg_part, x, W1)


def _mid_body(scat_ref, hwp_ref, dinvc_ref, al_ref, be_ref, w2_ref, out_ref):
    dv = dinvc_ref[...]
    acc = None
    for hh in range(2):
        t = (scat_ref[hh] + hwp_ref[hh]) * dv
        h1 = jnp.maximum(t * al_ref[hh] + be_ref[hh], 0.0)
        p = jnp.dot(h1, w2_ref[hh], preferred_element_type=jnp.float32, precision=lax.Precision.HIGHEST)
        acc = p if acc is None else acc + p
    out_ref[0] = acc * dv


def _run_mid(scat1, hwp1, dinvc, al1, be1, W2_rs):
    return pl.pallas_call(
        _mid_body,
        grid=(2, GR),
        in_specs=[
            pl.BlockSpec((2, R, DH), lambda h, r: (0, r, 0)),
            pl.BlockSpec((2, R, DH), lambda h, r: (0, r, 0)),
            pl.BlockSpec((R, DH), lambda h, r: (r, 0)),
            pl.BlockSpec((2, 128), lambda h, r: (0, 0)),
            pl.BlockSpec((2, 128), lambda h, r: (0, 0)),
            pl.BlockSpec((2, 128, 128), lambda h, r: (0, 0, h)),
        ],
        out_specs=pl.BlockSpec((1, R, DH), lambda h, r: (h, r, 0)),
        out_shape=jax.ShapeDtypeStruct((2, NPAD, DH), jnp.float32),
    )(scat1, hwp1, dinvc, al1, be1, W2_rs)


def _final_body(scat_ref, hwp_ref, dinvc_ref, al_ref, be_ref, deg_ref, s_ref,
                al3_ref, be3_ref, w3_ref, pw_ref, pb_ref, cw1_ref, cb1_ref,
                cw2_ref, cb2_ref, out_ref, acc_ref):
    r = pl.program_id(0)
    dv = dinvc_ref[...]
    row_ids = lax.broadcasted_iota(jnp.int32, (R, 1), 0) + r * R
    row_ok = row_ids < N

    parts = []
    for hh in range(2):
        t = (scat_ref[hh] + hwp_ref[hh]) * dv
        h2 = jnp.maximum(t * al_ref[hh] + be_ref[hh], 0.0)
        h2 = jnp.where(row_ok, h2, 0.0)
        parts.append(h2)

    deg_row = jnp.sum(deg_ref[...], axis=0, keepdims=True) + 1.0
    dinv_row = lax.rsqrt(deg_row)
    s_row = jnp.sum(s_ref[...], axis=0, keepdims=True)
    col_ids = lax.broadcasted_iota(jnp.int32, (1, R), 1) + r * R
    u = jnp.where(col_ids < N, dinv_row * (s_row + dinv_row), 0.0)
    u = u * (1.0 / N)

    part = jnp.concatenate(
        [jnp.dot(u, p, preferred_element_type=jnp.float32, precision=lax.Precision.HIGHEST) for p in parts],
        axis=1,
    )

    @pl.when(r == 0)
    def _():
        acc_ref[...] = part

    @pl.when(r > 0)
    def _():
        acc_ref[...] = acc_ref[...] + part

    @pl.when(r == GR - 1)
    def _():
        pooled = acc_ref[...]
        y = jnp.dot(pooled, w3_ref[...], preferred_element_type=jnp.float32, precision=lax.Precision.HIGHEST)
        z = y * al3_ref[...] + be3_ref[...]
        p = jnp.maximum(
            jnp.dot(z, pw_ref[...], preferred_element_type=jnp.float32, precision=lax.Precision.HIGHEST)
            + pb_ref[...], 0.0)
        cc = jnp.maximum(
            jnp.dot(p, cw1_ref[...], preferred_element_type=jnp.float32, precision=lax.Precision.HIGHEST)
            + cb1_ref[...], 0.0)
        out_ref[...] = (
            jnp.dot(cc, cw2_ref[...], preferred_element_type=jnp.float32, precision=lax.Precision.HIGHEST)
            + cb2_ref[...])


def _run_final(scat2, hwp2, dinvc, al2, be2, deg_part, s_part, al3, be3,
               W3, pW, pb, cW1, cb1, cW2, cb2):
    def full(shape):
        nz = len(shape)
        return pl.BlockSpec(shape, lambda r, _n=nz: (0,) * _n)

    return pl.pallas_call(
        _final_body,
        grid=(GR,),
        in_specs=[
            pl.BlockSpec((2, R, DH), lambda r: (0, r, 0)),
            pl.BlockSpec((2, R, DH), lambda r: (0, r, 0)),
            pl.BlockSpec((R, DH), lambda r: (r, 0)),
            full((2, 128)),
            full((2, 128)),
            pl.BlockSpec((NW, R), lambda r: (0, r)),
            pl.BlockSpec((NW, R), lambda r: (0, r)),
            full((1, D)),
            full((1, D)),
            full((D, D)),
            full((D, DH)),
            full((1, DH)),
            full((DH, 64)),
            full((1, 64)),
            full((64, DH)),
            full((1, DH)),
        ],
        out_specs=pl.BlockSpec((1, DH), lambda r: (0, 0)),
        out_shape=jax.ShapeDtypeStruct((1, DH), jnp.float32),
        scratch_shapes=[pltpu.VMEM((1, D), jnp.float32)],
    )(scat2, hwp2, dinvc, al2, be2, deg_part, s_part, al3, be3,
      W3, pW, pb, cW1, cb1, cW2, cb2)


def kernel(x, edge_index, W1, b1, g1, be1, m1, v1, W2, b2, g2, be2, m2, v2,
           W3, b3, g3, be3, m3, v3, pW, pb, cW1, cb1, cW2, cb2):
    def fold(b, g, be, m, v):
        scale = g * lax.rsqrt(v + _EPS)
        shift = b * scale + (be - m * scale)
        return scale, shift

    al1, bp1 = fold(b1, g1, be1, m1, v1)
    al2, bp2 = fold(b2, g2, be2, m2, v2)
    al3, bp3 = fold(b3, g3, be3, m3, v3)

    al1 = al1.reshape(2, 128)
    bp1 = bp1.reshape(2, 128)
    al2 = al2.reshape(2, 128)
    bp2 = bp2.reshape(2, 128)
    al3 = al3.reshape(1, D)
    bp3 = bp3.reshape(1, D)

    W2_rs = W2.reshape(2, 128, D)

    src_flat = edge_index[0]
    dst_flat = edge_index[1]
    pad_ids = jnp.arange(EP - E, dtype=jnp.int32)
    src_rs = jnp.concatenate(
        [src_flat, pad_ids % N]).reshape(NS, NCH, CH_R, K)
    dst_rs = jnp.concatenate(
        [dst_flat, N + pad_ids % (NPAD - N)]).reshape(NS, NCH, CH_R, K)

    deg_part = _run_deg(dst_flat)
    hwp1, dinvc, dinv2d = _run_hw1(deg_part, x, W1)
    dinv2d = dinv2d.reshape(NPAD // 128, 128)
    s_part = _run_s(src_flat, dst_flat, dinv2d)
    scat1 = _run_spmm(hwp1.reshape(2 * NPAD, DH), src_rs, dst_rs)
    hwp2 = _run_mid(scat1.reshape(2, NPAD, DH), hwp1, dinvc, al1, bp1, W2_rs)
    scat2 = _run_spmm(hwp2.reshape(2 * NPAD, DH), src_rs, dst_rs)
    out = _run_final(scat2.reshape(2, NPAD, DH), hwp2, dinvc, al2, bp2,
                     deg_part, s_part, al3, bp3, W3, pW,
                     pb.reshape(1, DH), cW1, cb1.reshape(1, 64),
                     cW2, cb2.reshape(1, DH))
    return out


# speedup vs baseline: 23.4280x; 1.0415x over previous
import functools

import jax
import jax.numpy as jnp
from jax import lax
from jax.experimental import pallas as pl
from jax.experimental.pallas import tpu as pltpu
from jax.experimental.pallas import tpu_sc as plsc

N = 10000
E = 160000
D = 256
DH = 128

NC = 2
NS = 16
NW = NC * NS

NPAD = 10240
R = 1280
GR = NPAD // R

K = 128
EPTP = 10240
EP = NS * EPTP
NBT = EPTP // K
NCH = 5
CH_R = NBT // NCH

EPW = E // NW
FULL_IT = EPW // 16
REM = EPW - FULL_IT * 16

STRIPE = NPAD // NS

_EPS = 1e-5


def _sc_mesh():
    return plsc.VectorSubcoreMesh(core_axis_name="c", subcore_axis_name="s")


def _deg_kernel(dst_flat, out, dst_v, acc):
    c = lax.axis_index("c")
    s = lax.axis_index("s")
    w = s * NC + c

    z16 = jnp.zeros((16,), jnp.float32)

    def zero_body(i, _):
        acc[pl.ds(i * 16, 16)] = z16
        return 0

    lax.fori_loop(0, NPAD // 16, zero_body, 0)
    dst_v[pl.ds(EPW, 16)] = jnp.zeros((16,), jnp.int32)
    pltpu.sync_copy(dst_flat.at[pl.ds(w * EPW, EPW)], dst_v.at[pl.ds(0, EPW)])

    ones = jnp.ones((16,), jnp.float32)

    def body(i, _):
        idx = dst_v[pl.ds(i * 16, 16)]
        plsc.addupdate_scatter(acc, [idx], ones)
        return 0

    lax.fori_loop(0, FULL_IT, body, 0)
    rem_mask = lax.broadcasted_iota(jnp.int32, (16,), 0) < REM
    idx = dst_v[pl.ds(FULL_IT * 16, 16)]
    plsc.addupdate_scatter(acc, [idx], ones, mask=rem_mask)

    pltpu.sync_copy(acc, out.at[w])


def _run_deg(dst_flat):
    f = pl.kernel(
        _deg_kernel,
        out_type=jax.ShapeDtypeStruct((NW, NPAD), jnp.float32),
        mesh=_sc_mesh(),
        compiler_params=pltpu.CompilerParams(needs_layout_passes=False),
        scratch_types=[
            pltpu.VMEM((EPW + 16,), jnp.int32),
            pltpu.VMEM((NPAD,), jnp.float32),
        ],
    )
    return f(dst_flat)


def _s_kernel(src_flat, dst_flat, dinv2d, out, src_v, dst_v, dinv_v, acc):
    c = lax.axis_index("c")
    s = lax.axis_index("s")
    w = s * NC + c

    z16 = jnp.zeros((16,), jnp.float32)

    def zero_body(i, _):
        acc[pl.ds(i * 16, 16)] = z16
        return 0

    lax.fori_loop(0, NPAD // 16, zero_body, 0)
    src_v[pl.ds(EPW, 16)] = jnp.zeros((16,), jnp.int32)
    dst_v[pl.ds(EPW, 16)] = jnp.zeros((16,), jnp.int32)
    pltpu.sync_copy(src_flat.at[pl.ds(w * EPW, EPW)], src_v.at[pl.ds(0, EPW)])
    pltpu.sync_copy(dst_flat.at[pl.ds(w * EPW, EPW)], dst_v.at[pl.ds(0, EPW)])
    pltpu.sync_copy(dinv2d, dinv_v)

    def step(i, mask):
        d_idx = dst_v[pl.ds(i * 16, 16)]
        val = plsc.load_gather(
            dinv_v,
            [lax.shift_right_logical(d_idx, 7), jnp.bitwise_and(d_idx, 127)],
            mask=mask,
        )
        s_idx = src_v[pl.ds(i * 16, 16)]
        plsc.addupdate_scatter(acc, [s_idx], val, mask=mask)

    full_mask = lax.broadcasted_iota(jnp.int32, (16,), 0) < 16

    def body(i, _):
        step(i, full_mask)
        return 0

    lax.fori_loop(0, FULL_IT, body, 0)
    rem_mask = lax.broadcasted_iota(jnp.int32, (16,), 0) < REM
    step(FULL_IT, rem_mask)

    pltpu.sync_copy(acc, out.at[w])


def _run_s(src_flat, dst_flat, dinv2d):
    f = pl.kernel(
        _s_kernel,
        out_type=jax.ShapeDtypeStruct((NW, NPAD), jnp.float32),
        mesh=_sc_mesh(),
        compiler_params=pltpu.CompilerParams(needs_layout_passes=False),
        scratch_types=[
            pltpu.VMEM((EPW + 16,), jnp.int32),
            pltpu.VMEM((EPW + 16,), jnp.int32),
            pltpu.VMEM((NPAD // 128, 128), jnp.float32),
            pltpu.VMEM((NPAD,), jnp.float32),
        ],
    )
    return f(src_flat, dst_flat, dinv2d)


def _spmm_kernel(hw_flat, src_rs, dst_rs, out, idxs_v, idxd_v, buf0, buf1,
                 acc, sems2):
    c = lax.axis_index("c")
    s = lax.axis_index("s")

    z16 = jnp.zeros((16,), jnp.float32)

    def zbuf_outer(j, _):
        def zbuf_inner(k, _2):
            buf0[j, pl.ds(k * 16, 16)] = z16
            return 0

        lax.fori_loop(0, DH // 16, zbuf_inner, 0)
        return 0

    lax.fori_loop(0, K, zbuf_outer, 0)

    for t in range(STRIPE // K):
        pltpu.sync_copy(buf0, acc.at[pl.ds(s * STRIPE + t * K, K)])
    plsc.subcore_barrier()

    offv = jnp.zeros((16,), jnp.int32) + (c * NPAD).astype(jnp.int32)
    bufs = (buf0, buf1)
    sems = (sems2.at[0], sems2.at[1])

    for ch in range(NCH):
        pltpu.sync_copy(src_rs.at[s, ch], idxs_v)
        pltpu.sync_copy(dst_rs.at[s, ch], idxd_v)

        def off_outer(j, _):
            def off_inner(k, _2):
                v = idxs_v[j, pl.ds(k * 16, 16)]
                idxs_v[j, pl.ds(k * 16, 16)] = v + offv
                return 0

            lax.fori_loop(0, K // 16, off_inner, 0)
            return 0

        lax.fori_loop(0, CH_R, off_outer, 0)

        pltpu.async_copy(hw_flat.at[idxs_v.at[0]], buf0, sems[0])

        @pl.loop(0, CH_R, step=2)
        def _(j):
            for b in range(2):
                nxt = j + b + 1

                @pl.when(nxt < CH_R)
                def _():
                    pltpu.async_copy(
                        hw_flat.at[idxs_v.at[nxt]], bufs[1 - b], sems[1 - b])

                pltpu.make_async_copy(
                    hw_flat.at[idxs_v.at[j + b]], bufs[b], sems[b]).wait()
                pltpu.sync_copy(bufs[b], acc.at[idxd_v.at[j + b]], add=True)

    plsc.subcore_barrier()

    pltpu.sync_copy(
        acc.at[pl.ds(s * STRIPE, STRIPE)],
        out.at[pl.ds(c * NPAD + s * STRIPE, STRIPE)],
    )


def _run_spmm(hw_flat, src_rs, dst_rs):
    f = pl.kernel(
        _spmm_kernel,
        out_type=jax.ShapeDtypeStruct((2 * NPAD, DH), jnp.float32),
        mesh=_sc_mesh(),
        compiler_params=pltpu.CompilerParams(needs_layout_passes=False),
        scratch_types=[
            pltpu.VMEM((CH_R, K), jnp.int32),
            pltpu.VMEM((CH_R, K), jnp.int32),
            pltpu.VMEM((K, DH), jnp.float32),
            pltpu.VMEM((K, DH), jnp.float32),
            pltpu.VMEM_SHARED((NPAD, DH), jnp.float32),
            pltpu.SemaphoreType.DMA((2,)),
        ],
    )
    return f(hw_flat, src_rs, dst_rs)


def _hw1_body(deg_ref, x_ref, w_ref, hwp_ref, dinvc_ref, dinv2d_ref):
    deg = jnp.sum(deg_ref[...], axis=0) + 1.0
    dinv_col = lax.rsqrt(deg).reshape(R, 1)
    hw = jnp.dot(x_ref[...], w_ref[...], preferred_element_type=jnp.float32)
    hwp_ref[0] = hw * dinv_col
    dinvc_ref[...] = jnp.broadcast_to(dinv_col, (R, DH))
    dinv2d_ref[...] = dinv_col.reshape(1, R // 128, 128)


def _run_hw1(deg_part, x, W1):
    return pl.pallas_call(
        _hw1_body,
        grid=(2, GR),
        in_specs=[
            pl.BlockSpec((NW, R), lambda h, r: (0, r)),
            pl.BlockSpec((R, D), lambda h, r: (r, 0)),
            pl.BlockSpec((D, DH), lambda h, r: (0, h)),
        ],
        out_specs=[
            pl.BlockSpec((1, R, DH), lambda h, r: (h, r, 0)),
            pl.BlockSpec((R, DH), lambda h, r: (r, 0)),
            pl.BlockSpec((1, R // 128, 128), lambda h, r: (r, 0, 0)),
        ],
        out_shape=[
            jax.ShapeDtypeStruct((2, NPAD, DH), jnp.float32),
            jax.ShapeDtypeStruct((NPAD, DH), jnp.float32),
            jax.ShapeDtypeStruct((GR, R // 128, 128), jnp.float32),
        ],
    )(deg_part, x, W1)


def _mid_body(scat_ref, hwp_ref, dinvc_ref, al_ref, be_ref, w2_ref, out_ref):
    dv = dinvc_ref[...]
    acc = None
    for hh in range(2):
        t = (scat_ref[hh] + hwp_ref[hh]) * dv
        h1 = jnp.maximum(t * al_ref[hh] + be_ref[hh], 0.0)
        p = jnp.dot(h1, w2_ref[hh], preferred_element_type=jnp.float32)
        acc = p if acc is None else acc + p
    out_ref[0] = acc * dv


def _run_mid(scat1, hwp1, dinvc, al1, be1, W2_rs):
    return pl.pallas_call(
        _mid_body,
        grid=(2, GR),
        in_specs=[
            pl.BlockSpec((2, R, DH), lambda h, r: (0, r, 0)),
            pl.BlockSpec((2, R, DH), lambda h, r: (0, r, 0)),
            pl.BlockSpec((R, DH), lambda h, r: (r, 0)),
            pl.BlockSpec((2, 128), lambda h, r: (0, 0)),
            pl.BlockSpec((2, 128), lambda h, r: (0, 0)),
            pl.BlockSpec((2, 128, 128), lambda h, r: (0, 0, h)),
        ],
        out_specs=pl.BlockSpec((1, R, DH), lambda h, r: (h, r, 0)),
        out_shape=jax.ShapeDtypeStruct((2, NPAD, DH), jnp.float32),
    )(scat1, hwp1, dinvc, al1, be1, W2_rs)


def _final_body(scat_ref, hwp_ref, dinvc_ref, al_ref, be_ref, deg_ref, s_ref,
                al3_ref, be3_ref, w3_ref, pw_ref, pb_ref, cw1_ref, cb1_ref,
                cw2_ref, cb2_ref, out_ref, acc_ref):
    r = pl.program_id(0)
    dv = dinvc_ref[...]
    row_ids = lax.broadcasted_iota(jnp.int32, (R, 1), 0) + r * R
    row_ok = row_ids < N

    parts = []
    for hh in range(2):
        t = (scat_ref[hh] + hwp_ref[hh]) * dv
        h2 = jnp.maximum(t * al_ref[hh] + be_ref[hh], 0.0)
        h2 = jnp.where(row_ok, h2, 0.0)
        parts.append(h2)

    deg_row = jnp.sum(deg_ref[...], axis=0, keepdims=True) + 1.0
    dinv_row = lax.rsqrt(deg_row)
    s_row = jnp.sum(s_ref[...], axis=0, keepdims=True)
    col_ids = lax.broadcasted_iota(jnp.int32, (1, R), 1) + r * R
    u = jnp.where(col_ids < N, dinv_row * (s_row + dinv_row), 0.0)
    u = u * (1.0 / N)

    part = jnp.concatenate(
        [jnp.dot(u, p, preferred_element_type=jnp.float32, precision=lax.Precision.HIGHEST) for p in parts],
        axis=1,
    )

    @pl.when(r == 0)
    def _():
        acc_ref[...] = part

    @pl.when(r > 0)
    def _():
        acc_ref[...] = acc_ref[...] + part

    @pl.when(r == GR - 1)
    def _():
        pooled = acc_ref[...]
        y = jnp.dot(pooled, w3_ref[...], preferred_element_type=jnp.float32, precision=lax.Precision.HIGHEST)
        z = y * al3_ref[...] + be3_ref[...]
        p = jnp.maximum(
            jnp.dot(z, pw_ref[...], preferred_element_type=jnp.float32, precision=lax.Precision.HIGHEST)
            + pb_ref[...], 0.0)
        cc = jnp.maximum(
            jnp.dot(p, cw1_ref[...], preferred_element_type=jnp.float32, precision=lax.Precision.HIGHEST)
            + cb1_ref[...], 0.0)
        out_ref[...] = (
            jnp.dot(cc, cw2_ref[...], preferred_element_type=jnp.float32, precision=lax.Precision.HIGHEST)
            + cb2_ref[...])


def _run_final(scat2, hwp2, dinvc, al2, be2, deg_part, s_part, al3, be3,
               W3, pW, pb, cW1, cb1, cW2, cb2):
    def full(shape):
        nz = len(shape)
        return pl.BlockSpec(shape, lambda r, _n=nz: (0,) * _n)

    return pl.pallas_call(
        _final_body,
        grid=(GR,),
        in_specs=[
            pl.BlockSpec((2, R, DH), lambda r: (0, r, 0)),
            pl.BlockSpec((2, R, DH), lambda r: (0, r, 0)),
            pl.BlockSpec((R, DH), lambda r: (r, 0)),
            full((2, 128)),
            full((2, 128)),
            pl.BlockSpec((NW, R), lambda r: (0, r)),
            pl.BlockSpec((NW, R), lambda r: (0, r)),
            full((1, D)),
            full((1, D)),
            full((D, D)),
            full((D, DH)),
            full((1, DH)),
            full((DH, 64)),
            full((1, 64)),
            full((64, DH)),
            full((1, DH)),
        ],
        out_specs=pl.BlockSpec((1, DH), lambda r: (0, 0)),
        out_shape=jax.ShapeDtypeStruct((1, DH), jnp.float32),
        scratch_shapes=[pltpu.VMEM((1, D), jnp.float32)],
    )(scat2, hwp2, dinvc, al2, be2, deg_part, s_part, al3, be3,
      W3, pW, pb, cW1, cb1, cW2, cb2)


def kernel(x, edge_index, W1, b1, g1, be1, m1, v1, W2, b2, g2, be2, m2, v2,
           W3, b3, g3, be3, m3, v3, pW, pb, cW1, cb1, cW2, cb2):
    def fold(b, g, be, m, v):
        scale = g * lax.rsqrt(v + _EPS)
        shift = b * scale + (be - m * scale)
        return scale, shift

    al1, bp1 = fold(b1, g1, be1, m1, v1)
    al2, bp2 = fold(b2, g2, be2, m2, v2)
    al3, bp3 = fold(b3, g3, be3, m3, v3)

    al1 = al1.reshape(2, 128)
    bp1 = bp1.reshape(2, 128)
    al2 = al2.reshape(2, 128)
    bp2 = bp2.reshape(2, 128)
    al3 = al3.reshape(1, D)
    bp3 = bp3.reshape(1, D)

    W2_rs = W2.reshape(2, 128, D)

    src_flat = edge_index[0]
    dst_flat = edge_index[1]
    pad_ids = jnp.arange(EP - E, dtype=jnp.int32)
    src_rs = jnp.concatenate(
        [src_flat, pad_ids % N]).reshape(NS, NCH, CH_R, K)
    dst_rs = jnp.concatenate(
        [dst_flat, N + pad_ids % (NPAD - N)]).reshape(NS, NCH, CH_R, K)

    deg_part = _run_deg(dst_flat)
    hwp1, dinvc, dinv2d = _run_hw1(deg_part, x, W1)
    dinv2d = dinv2d.reshape(NPAD // 128, 128)
    s_part = _run_s(src_flat, dst_flat, dinv2d)
    scat1 = _run_spmm(hwp1.reshape(2 * NPAD, DH), src_rs, dst_rs)
    hwp2 = _run_mid(scat1.reshape(2, NPAD, DH), hwp1, dinvc, al1, bp1, W2_rs)
    scat2 = _run_spmm(hwp2.reshape(2 * NPAD, DH), src_rs, dst_rs)
    out = _run_final(scat2.reshape(2, NPAD, DH), hwp2, dinvc, al2, bp2,
                     deg_part, s_part, al3, bp3, W3, pW,
                     pb.reshape(1, DH), cW1, cb1.reshape(1, 64),
                     cW2, cb2.reshape(1, DH))
    return out
